# Initial kernel scaffold; baseline (speedup 1.0000x reference)
import jax
import jax.numpy as jnp
from jax.experimental import pallas as pl


def kernel(cls_output, reg_output, anchors):
    raise NotImplementedError("write your pallas kernel here")


# speedup vs baseline: 17.4256x; 17.4256x over previous
import jax
import jax.numpy as jnp
from jax.experimental import pallas as pl
from jax.experimental.pallas import tpu as pltpu

_N = 22500
_ROWS = 176
_NPAD = _ROWS * 128
_MAX_OUT = 300
_IOU_THR = 0.7
_IMG = 800.0


def _nms_body(scores_ref, reg_ref, anc_ref, out_ref, run_ref, box_ref):
    f0 = jnp.float32(0.0)
    x1a = anc_ref[0]
    y1a = anc_ref[1]
    x2a = anc_ref[2]
    y2a = anc_ref[3]
    wa = x2a - x1a
    ha = y2a - y1a
    cxa = x1a + wa * 0.5
    cya = y1a + ha * 0.5
    cx = reg_ref[0] * wa + cxa
    cy = reg_ref[1] * ha + cya
    w = wa * jnp.exp(reg_ref[2])
    h = ha * jnp.exp(reg_ref[3])
    x1 = jnp.minimum(jnp.maximum(cx - w * 0.5, f0), _IMG)
    y1 = jnp.minimum(jnp.maximum(cy - h * 0.5, f0), _IMG)
    x2 = jnp.minimum(jnp.maximum(cx + w * 0.5, f0), _IMG)
    y2 = jnp.minimum(jnp.maximum(cy + h * 0.5, f0), _IMG)
    box_ref[0] = x1
    box_ref[1] = y1
    box_ref[2] = x2
    box_ref[3] = y2
    box_ref[4] = (x2 - x1) * (y2 - y1)

    run_ref[...] = scores_ref[...]

    rows = jax.lax.broadcasted_iota(jnp.int32, (_ROWS, 128), 0)
    cols = jax.lax.broadcasted_iota(jnp.int32, (_ROWS, 128), 1)
    lin = rows * 128 + cols
    lane = jax.lax.broadcasted_iota(jnp.int32, (1, 128), 1)

    def step(i, _):
        s = run_ref[...]
        m = jnp.max(s)
        idx = jnp.min(jnp.where(s == m, lin, _NPAD))
        selm = lin == idx
        x1c = box_ref[0]
        y1c = box_ref[1]
        x2c = box_ref[2]
        y2c = box_ref[3]
        x1s = jnp.sum(jnp.where(selm, x1c, f0))
        y1s = jnp.sum(jnp.where(selm, y1c, f0))
        x2s = jnp.sum(jnp.where(selm, x2c, f0))
        y2s = jnp.sum(jnp.where(selm, y2c, f0))
        ss = jnp.sum(jnp.where(selm, scores_ref[...], f0))
        area_s = (x2s - x1s) * (y2s - y1s)
        xx1 = jnp.maximum(x1c, x1s)
        yy1 = jnp.maximum(y1c, y1s)
        xx2 = jnp.minimum(x2c, x2s)
        yy2 = jnp.minimum(y2c, y2s)
        inter = jnp.maximum(xx2 - xx1, f0) * jnp.maximum(yy2 - yy1, f0)
        iou = inter / (box_ref[4] + area_s - inter + 1e-9)
        run_ref[...] = jnp.where(iou >= _IOU_THR, -1e9, s)
        row = jnp.where(lane == 0, x1s,
              jnp.where(lane == 1, y1s,
              jnp.where(lane == 2, x2s,
              jnp.where(lane == 3, y2s, ss))))
        out_ref[pl.ds(i, 1), :] = row
        return 0

    jax.lax.fori_loop(0, _MAX_OUT, step, 0)


def kernel(cls_output, reg_output, anchors):
    f32 = jnp.float32
    pad = _NPAD - _N
    scores = jnp.concatenate(
        [cls_output.astype(f32), jnp.full((pad,), -jnp.inf, f32)]
    ).reshape(_ROWS, 128)
    reg4 = jnp.concatenate(
        [reg_output.astype(f32), jnp.zeros((pad, 4), f32)]
    ).T.reshape(4, _ROWS, 128)
    anc4 = jnp.concatenate(
        [anchors.astype(f32), jnp.zeros((pad, 4), f32)]
    ).T.reshape(4, _ROWS, 128)

    out = pl.pallas_call(
        _nms_body,
        out_shape=jax.ShapeDtypeStruct((304, 128), f32),
        scratch_shapes=[
            pltpu.VMEM((_ROWS, 128), f32),
            pltpu.VMEM((5, _ROWS, 128), f32),
        ],
    )(scores, reg4, anc4)

    rois = out[:_MAX_OUT, 0:4]
    roi_scores = out[:_MAX_OUT, 4]
    return roi_scores, rois


# speedup vs baseline: 18.7581x; 1.0765x over previous
import jax
import jax.numpy as jnp
from jax.experimental import pallas as pl
from jax.experimental.pallas import tpu as pltpu

_N = 22500
_ROWS = 176
_NPAD = _ROWS * 128
_MAX_OUT = 300
_IOU_THR = 0.7
_IMG = 800.0


def _nms_body(scores_ref, reg_ref, anc_ref, out_ref, run_ref, box_ref):
    f0 = jnp.float32(0.0)
    x1a = anc_ref[0]
    y1a = anc_ref[1]
    x2a = anc_ref[2]
    y2a = anc_ref[3]
    wa = x2a - x1a
    ha = y2a - y1a
    cxa = x1a + wa * 0.5
    cya = y1a + ha * 0.5
    cx = reg_ref[0] * wa + cxa
    cy = reg_ref[1] * ha + cya
    w = wa * jnp.exp(reg_ref[2])
    h = ha * jnp.exp(reg_ref[3])
    x1 = jnp.minimum(jnp.maximum(cx - w * 0.5, f0), _IMG)
    y1 = jnp.minimum(jnp.maximum(cy - h * 0.5, f0), _IMG)
    x2 = jnp.minimum(jnp.maximum(cx + w * 0.5, f0), _IMG)
    y2 = jnp.minimum(jnp.maximum(cy + h * 0.5, f0), _IMG)
    box_ref[0] = x1
    box_ref[1] = y1
    box_ref[2] = x2
    box_ref[3] = y2
    box_ref[4] = (x2 - x1) * (y2 - y1)

    run_ref[...] = scores_ref[...]

    rows = jax.lax.broadcasted_iota(jnp.int32, (_ROWS, 128), 0)
    cols = jax.lax.broadcasted_iota(jnp.int32, (_ROWS, 128), 1)
    lin = rows * 128 + cols
    lane = jax.lax.broadcasted_iota(jnp.int32, (1, 128), 1)

    def step(i, m):
        s = run_ref[...]
        idx = jnp.min(jnp.where(s == m, lin, _NPAD))
        r = idx // 128
        c = idx - r * 128
        lsel = lane == c
        x1s = jnp.sum(jnp.where(lsel, box_ref[0, pl.ds(r, 1), :], f0))
        y1s = jnp.sum(jnp.where(lsel, box_ref[1, pl.ds(r, 1), :], f0))
        x2s = jnp.sum(jnp.where(lsel, box_ref[2, pl.ds(r, 1), :], f0))
        y2s = jnp.sum(jnp.where(lsel, box_ref[3, pl.ds(r, 1), :], f0))
        ss = jnp.sum(jnp.where(lsel, scores_ref[pl.ds(r, 1), :], f0))
        area_s = (x2s - x1s) * (y2s - y1s)
        xx1 = jnp.maximum(box_ref[0], x1s)
        yy1 = jnp.maximum(box_ref[1], y1s)
        xx2 = jnp.minimum(box_ref[2], x2s)
        yy2 = jnp.minimum(box_ref[3], y2s)
        inter = jnp.maximum(xx2 - xx1, f0) * jnp.maximum(yy2 - yy1, f0)
        iou = inter / (box_ref[4] + area_s - inter + 1e-9)
        s2 = jnp.where(iou >= _IOU_THR, -1e9, s)
        run_ref[...] = s2
        row = jnp.where(lane == 0, x1s,
              jnp.where(lane == 1, y1s,
              jnp.where(lane == 2, x2s,
              jnp.where(lane == 3, y2s, ss))))
        out_ref[pl.ds(i, 1), :] = row
        return jnp.max(s2)

    jax.lax.fori_loop(0, _MAX_OUT, step, jnp.max(scores_ref[...]))


def kernel(cls_output, reg_output, anchors):
    f32 = jnp.float32
    pad = _NPAD - _N
    scores = jnp.concatenate(
        [cls_output.astype(f32), jnp.full((pad,), -jnp.inf, f32)]
    ).reshape(_ROWS, 128)
    reg4 = jnp.concatenate(
        [reg_output.astype(f32), jnp.zeros((pad, 4), f32)]
    ).T.reshape(4, _ROWS, 128)
    anc4 = jnp.concatenate(
        [anchors.astype(f32), jnp.zeros((pad, 4), f32)]
    ).T.reshape(4, _ROWS, 128)

    out = pl.pallas_call(
        _nms_body,
        out_shape=jax.ShapeDtypeStruct((304, 128), f32),
        scratch_shapes=[
            pltpu.VMEM((_ROWS, 128), f32),
            pltpu.VMEM((5, _ROWS, 128), f32),
        ],
    )(scores, reg4, anc4)

    rois = out[:_MAX_OUT, 0:4]
    roi_scores = out[:_MAX_OUT, 4]
    return roi_scores, rois
